# baseline (speedup 1.0000x reference)
import functools

import jax
import jax.numpy as jnp
from jax import lax
from jax.experimental import pallas as pl
from jax.experimental.pallas import tpu as pltpu
from jax.experimental.pallas import tpu_sc as plsc

_N = 10000
_E = 320000
_DIN = 128
_DH = 128
_DC = 64

_NC = 2
_NS = 16
_NW = _NC * _NS
_B = 128
_KCH = 80
_EPAD = _NW * _KCH * _B
_NPAD = 10240
_RPT = _NPAD // _NS

_SELU_ALPHA = 1.6732632423543772
_SELU_SCALE = 1.0507009873554805


def _deg_body(dst_hbm, zeros_hbm, degp_hbm, dst_v, deg_v):
    wid = lax.axis_index("s") * _NC + lax.axis_index("c")
    pltpu.sync_copy(zeros_hbm.at[pl.ds(0, 80)], deg_v)
    pltpu.sync_copy(dst_hbm.at[pl.ds(wid * _KCH, _KCH)], dst_v)
    ones16 = jnp.ones((16,), jnp.float32)

    def step(j, carry):
        for k in range(_B // 16):
            idx = dst_v[j, pl.ds(k * 16, 16)]
            row = lax.shift_right_logical(idx, 7)
            col = lax.bitwise_and(idx, 127)
            plsc.addupdate_scatter(deg_v, [row, col], ones16)
        return carry

    lax.fori_loop(0, _KCH, step, 0)
    pltpu.sync_copy(deg_v, degp_hbm.at[wid])


def _deg_kernel(dst_r, zeros2d):
    return pl.kernel(
        _deg_body,
        out_type=jax.ShapeDtypeStruct((_NW, 80, 128), jnp.float32),
        mesh=plsc.VectorSubcoreMesh(core_axis_name="c", subcore_axis_name="s",
                                    num_cores=_NC, num_subcores=_NS),
        scratch_types=[
            pltpu.VMEM((_KCH, _B), jnp.int32),
            pltpu.VMEM((80, 128), jnp.float32),
        ],
        compiler_params=pltpu.CompilerParams(needs_layout_passes=False),
    )(dst_r, zeros2d)


_NCHUNK = _EPAD // _B
_K0 = 160
_K1 = 0
_KST = 16


def _seg_body(src_hbm, dst_hbm, feat_hbm, outp_hbm,
              src_v, dst_v, r0, acc_sh, s0, s1):
    cid = lax.axis_index("c")
    sid = lax.axis_index("s")
    sems = (s0, s1)

    zero16 = jnp.zeros((16,), jnp.float32)
    dwords = r0.shape[1] // 16

    def zstep(i, carry):
        for k in range(dwords):
            r0[i, pl.ds(k * 16, 16)] = zero16
        return carry

    lax.fori_loop(0, 2 * _B, zstep, 0)
    for t in range(_RPT // (2 * _B)):
        pltpu.sync_copy(r0, acc_sh.at[pl.ds(sid * _RPT + t * 2 * _B, 2 * _B)])
    rem = _RPT % (2 * _B)
    if rem:
        pltpu.sync_copy(
            r0.at[pl.ds(0, rem)],
            acc_sh.at[pl.ds(sid * _RPT + (_RPT // (2 * _B)) * 2 * _B, rem)])
    plsc.subcore_barrier()

    cb = jnp.where(cid == 0, sid * _K0, 16 * _K0 + sid * _K1)
    nst = jnp.where(cid == 0, _K0 // _KST, _K1 // _KST)

    def gath(j, b):
        return pltpu.make_async_copy(
            feat_hbm.at[src_v.at[j]], r0.at[pl.ds(b * _B, _B)], sems[b])

    def stage(st, carry):
        base = cb + st * _KST
        pltpu.sync_copy(src_hbm.at[pl.ds(base, _KST)], src_v)
        pltpu.sync_copy(dst_hbm.at[pl.ds(base, _KST)], dst_v)
        for b in range(2):
            gath(b, b).start()

        def group(g, c):
            for b in range(2):
                j = 2 * g + b
                gath(j, b).wait()
                pltpu.sync_copy(r0.at[pl.ds(b * _B, _B)],
                                acc_sh.at[dst_v.at[j]], add=True)
                gath(jnp.minimum(j + 2, _KST - 1), b).start()
            return c

        lax.fori_loop(0, _KST // 2, group, 0)
        for b in range(2):
            gath(_KST - 1, b).wait()
        return carry

    lax.fori_loop(0, nst, stage, 0)
    plsc.subcore_barrier()
    pltpu.sync_copy(acc_sh.at[pl.ds(sid * _RPT, _RPT)],
                    outp_hbm.at[cid, pl.ds(sid * _RPT, _RPT)])


def _seg_call(src_r, dst_r, feat, d):
    return pl.kernel(
        _seg_body,
        out_type=jax.ShapeDtypeStruct((_NC, _NPAD, d), jnp.float32),
        mesh=plsc.VectorSubcoreMesh(core_axis_name="c", subcore_axis_name="s",
                                    num_cores=_NC, num_subcores=_NS),
        scratch_types=[
            pltpu.VMEM((_KST, _B), jnp.int32),
            pltpu.VMEM((_KST, _B), jnp.int32),
            pltpu.VMEM((2 * _B, d), jnp.float32),
            pltpu.VMEM_SHARED((_NPAD, d), jnp.float32),
            pltpu.SemaphoreType.DMA,
            pltpu.SemaphoreType.DMA,
        ],
    )(src_r, dst_r, feat)


_BLK = _NPAD


def _t0_body(degp_ref, x_ref, d_ref, ci_ref, g0_ref):
    deg = jnp.sum(degp_ref[...], axis=0)
    degc = jnp.maximum(deg, 1.0)
    d = jnp.where(deg > 0, lax.rsqrt(degc), 0.0)
    d_ref[...] = d[:, None]
    ci_ref[...] = (1.0 / degc)[:, None]
    g0_ref[...] = x_ref[...] * d[:, None]


def _t0(degp, xp):
    return pl.pallas_call(
        _t0_body,
        grid=(_NPAD // _BLK,),
        in_specs=[
            pl.BlockSpec((_NW, _BLK), lambda i: (0, i)),
            pl.BlockSpec((_BLK, _DIN), lambda i: (i, 0)),
        ],
        out_specs=[
            pl.BlockSpec((_BLK, 1), lambda i: (i, 0)),
            pl.BlockSpec((_BLK, 1), lambda i: (i, 0)),
            pl.BlockSpec((_BLK, _DIN), lambda i: (i, 0)),
        ],
        out_shape=[
            jax.ShapeDtypeStruct((_NPAD, 1), jnp.float32),
            jax.ShapeDtypeStruct((_NPAD, 1), jnp.float32),
            jax.ShapeDtypeStruct((_NPAD, _DIN), jnp.float32),
        ],
    )(degp, xp)


def _prop_body(scale_g, a0_ref, a1_ref, d_ref, prev_ref, h_ref, g_ref):
    d = d_ref[...]
    h = d * (a0_ref[...] + a1_ref[...]) + prev_ref[...]
    h_ref[...] = h
    g_ref[...] = (d * h) if scale_g else h


def _prop(a, d, prev, scale_g):
    return pl.pallas_call(
        functools.partial(_prop_body, scale_g),
        grid=(_NPAD // _BLK,),
        in_specs=[pl.BlockSpec((_BLK, _DH), lambda i: (i, 0)),
                  pl.BlockSpec((_BLK, _DH), lambda i: (i, 0)),
                  pl.BlockSpec((_BLK, 1), lambda i: (i, 0)),
                  pl.BlockSpec((_BLK, _DH), lambda i: (i, 0))],
        out_specs=[pl.BlockSpec((_BLK, _DH), lambda i: (i, 0))] * 2,
        out_shape=[jax.ShapeDtypeStruct((_NPAD, _DH), jnp.float32)] * 2,
    )(a[0], a[1], d, prev)


def _selu(z):
    return _SELU_SCALE * jnp.where(z > 0, z, _SELU_ALPHA * (jnp.exp(z) - 1.0))


def _t3_body(s0_ref, s1_ref, ci_ref, h2_ref, w1lt_ref, b1_ref, w1rt_ref,
             b2_ref, w2rt_ref, y_ref, r_ref):
    m1 = (s0_ref[...] + s1_ref[...]) * ci_ref[...]
    h2 = h2_ref[...]
    z = (jnp.dot(m1, w1lt_ref[...], preferred_element_type=jnp.float32)
         + jnp.dot(h2, w1rt_ref[...], preferred_element_type=jnp.float32)
         + b1_ref[...])
    y = _selu(z)
    y_ref[...] = y
    r_ref[...] = (jnp.dot(y, w2rt_ref[...], preferred_element_type=jnp.float32)
                  + b2_ref[...])


def _t3(s0, s1, ci, h2, w1lt, b1, w1rt, b2, w2rt):
    full = lambda shape: pl.BlockSpec(shape, lambda i: tuple(0 for _ in shape))
    return pl.pallas_call(
        _t3_body,
        grid=(_NPAD // _BLK,),
        in_specs=[
            pl.BlockSpec((_BLK, _DH), lambda i: (i, 0)),
            pl.BlockSpec((_BLK, _DH), lambda i: (i, 0)),
            pl.BlockSpec((_BLK, 1), lambda i: (i, 0)),
            pl.BlockSpec((_BLK, _DH), lambda i: (i, 0)),
            full((_DIN, _DH)),
            full((1, _DH)),
            full((_DIN, _DH)),
            full((1, _DC)),
            full((_DH, _DC)),
        ],
        out_specs=[pl.BlockSpec((_BLK, _DH), lambda i: (i, 0)),
                   pl.BlockSpec((_BLK, _DC), lambda i: (i, 0))],
        out_shape=[jax.ShapeDtypeStruct((_NPAD, _DH), jnp.float32),
                   jax.ShapeDtypeStruct((_NPAD, _DC), jnp.float32)],
    )(s0, s1, ci, h2, w1lt, b1, w1rt, b2, w2rt)


def _t4_body(s0_ref, s1_ref, ci_ref, r_ref, w2lt_ref, o_ref):
    m2 = (s0_ref[...] + s1_ref[...]) * ci_ref[...]
    t = (jnp.dot(m2, w2lt_ref[...], preferred_element_type=jnp.float32)
         + r_ref[...])
    m = jnp.max(t, axis=1, keepdims=True)
    e = jnp.exp(t - m)
    o_ref[...] = e / jnp.sum(e, axis=1, keepdims=True)


def _t4(s0, s1, ci, r, w2lt):
    return pl.pallas_call(
        _t4_body,
        grid=(_NPAD // _BLK,),
        in_specs=[
            pl.BlockSpec((_BLK, _DH), lambda i: (i, 0)),
            pl.BlockSpec((_BLK, _DH), lambda i: (i, 0)),
            pl.BlockSpec((_BLK, 1), lambda i: (i, 0)),
            pl.BlockSpec((_BLK, _DC), lambda i: (i, 0)),
            pl.BlockSpec((_DH, _DC), lambda i: (0, 0)),
        ],
        out_specs=pl.BlockSpec((_BLK, _DC), lambda i: (i, 0)),
        out_shape=jax.ShapeDtypeStruct((_NPAD, _DC), jnp.float32),
    )(s0, s1, ci, r, w2lt)


def kernel(x, edge_index, W1l, b1, W1r, W2l, b2, W2r):
    src = edge_index[0]
    dst = edge_index[1]
    padv = jnp.full((_EPAD - _E,), _N, dtype=jnp.int32)
    src_r = jnp.concatenate([src, padv]).reshape(_NCHUNK, _B)
    dst_r = jnp.concatenate([dst, padv]).reshape(_NCHUNK, _B)
    xp = jnp.pad(x, ((0, _NPAD - _N), (0, 0)))

    zeros2d = jnp.zeros((80, 128), jnp.float32)

    degp = _deg_kernel(dst_r, zeros2d).reshape(_NW, _NPAD)
    d, ci, g0 = _t0(degp, xp)

    a1 = _seg_call(src_r, dst_r, g0, _DH)
    h1, g1 = _prop(a1, d, xp, True)
    a2 = _seg_call(src_r, dst_r, g1, _DH)
    h2, h2b = _prop(a2, d, h1, False)

    s1 = _seg_call(src_r, dst_r, h2b, _DH)
    y, r = _t3(s1[0], s1[1], ci, h2b,
               W1l.T, b1.reshape(1, _DH), W1r.T,
               b2.reshape(1, _DC), W2r.T)
    s2 = _seg_call(src_r, dst_r, y, _DH)
    out = _t4(s2[0], s2[1], ci, r, W2l.T)
    return out[:_N]


# speedup vs baseline: 1.3455x; 1.3455x over previous
import functools

import jax
import jax.numpy as jnp
from jax import lax
from jax.experimental import pallas as pl
from jax.experimental.pallas import tpu as pltpu
from jax.experimental.pallas import tpu_sc as plsc

_N = 10000
_E = 320000
_DIN = 128
_DH = 128
_DC = 64

_NC = 2
_NS = 16
_NW = _NC * _NS
_B = 128
_KCH = 80
_EPAD = _NW * _KCH * _B
_NPAD = 10240
_RPT = _NPAD // _NS

_SELU_ALPHA = 1.6732632423543772
_SELU_SCALE = 1.0507009873554805


def _deg_body(dst_hbm, zeros_hbm, degp_hbm, dst_v, deg_v):
    wid = lax.axis_index("s") * _NC + lax.axis_index("c")
    pltpu.sync_copy(zeros_hbm.at[pl.ds(0, 80)], deg_v)
    pltpu.sync_copy(dst_hbm.at[pl.ds(wid * _KCH, _KCH)], dst_v)
    ones16 = jnp.ones((16,), jnp.float32)

    def step(j, carry):
        for k in range(_B // 16):
            idx = dst_v[j, pl.ds(k * 16, 16)]
            row = lax.shift_right_logical(idx, 7)
            col = lax.bitwise_and(idx, 127)
            plsc.addupdate_scatter(deg_v, [row, col], ones16)
        return carry

    lax.fori_loop(0, _KCH, step, 0)
    pltpu.sync_copy(deg_v, degp_hbm.at[wid])


def _deg_kernel(dst_r, zeros2d):
    return pl.kernel(
        _deg_body,
        out_type=jax.ShapeDtypeStruct((_NW, 80, 128), jnp.float32),
        mesh=plsc.VectorSubcoreMesh(core_axis_name="c", subcore_axis_name="s",
                                    num_cores=_NC, num_subcores=_NS),
        scratch_types=[
            pltpu.VMEM((_KCH, _B), jnp.int32),
            pltpu.VMEM((80, 128), jnp.float32),
        ],
        compiler_params=pltpu.CompilerParams(needs_layout_passes=False),
    )(dst_r, zeros2d)


_NCHUNK = _EPAD // _B
_K0 = 144
_K1 = 16
_KST = 16


def _seg_body(src_hbm, dst_hbm, feat_hbm, outp_hbm,
              src_v, dst_v, r0, acc_sh, s0, s1):
    cid = lax.axis_index("c")
    sid = lax.axis_index("s")
    sems = (s0, s1)

    zero16 = jnp.zeros((16,), jnp.float32)
    dwords = r0.shape[1] // 16

    def zstep(i, carry):
        for k in range(dwords):
            r0[i, pl.ds(k * 16, 16)] = zero16
        return carry

    lax.fori_loop(0, 2 * _B, zstep, 0)
    for t in range(_RPT // (2 * _B)):
        pltpu.sync_copy(r0, acc_sh.at[pl.ds(sid * _RPT + t * 2 * _B, 2 * _B)])
    rem = _RPT % (2 * _B)
    if rem:
        pltpu.sync_copy(
            r0.at[pl.ds(0, rem)],
            acc_sh.at[pl.ds(sid * _RPT + (_RPT // (2 * _B)) * 2 * _B, rem)])
    plsc.subcore_barrier()

    cb = jnp.where(cid == 0, sid * _K0, 16 * _K0 + sid * _K1)
    nst = jnp.where(cid == 0, _K0 // _KST, _K1 // _KST)

    def gath(j, b):
        return pltpu.make_async_copy(
            feat_hbm.at[src_v.at[j]], r0.at[pl.ds(b * _B, _B)], sems[b])

    def stage(st, carry):
        base = cb + st * _KST
        pltpu.sync_copy(src_hbm.at[pl.ds(base, _KST)], src_v)
        pltpu.sync_copy(dst_hbm.at[pl.ds(base, _KST)], dst_v)
        for b in range(2):
            gath(b, b).start()

        def group(g, c):
            for b in range(2):
                j = 2 * g + b
                gath(j, b).wait()
                pltpu.sync_copy(r0.at[pl.ds(b * _B, _B)],
                                acc_sh.at[dst_v.at[j]], add=True)
                gath(jnp.minimum(j + 2, _KST - 1), b).start()
            return c

        lax.fori_loop(0, _KST // 2, group, 0)
        for b in range(2):
            gath(_KST - 1, b).wait()
        return carry

    lax.fori_loop(0, nst, stage, 0)
    plsc.subcore_barrier()
    pltpu.sync_copy(acc_sh.at[pl.ds(sid * _RPT, _RPT)],
                    outp_hbm.at[cid, pl.ds(sid * _RPT, _RPT)])


def _seg_call(src_r, dst_r, feat, d):
    return pl.kernel(
        _seg_body,
        out_type=jax.ShapeDtypeStruct((_NC, _NPAD, d), jnp.float32),
        mesh=plsc.VectorSubcoreMesh(core_axis_name="c", subcore_axis_name="s",
                                    num_cores=_NC, num_subcores=_NS),
        scratch_types=[
            pltpu.VMEM((_KST, _B), jnp.int32),
            pltpu.VMEM((_KST, _B), jnp.int32),
            pltpu.VMEM((2 * _B, d), jnp.float32),
            pltpu.VMEM_SHARED((_NPAD, d), jnp.float32),
            pltpu.SemaphoreType.DMA,
            pltpu.SemaphoreType.DMA,
        ],
    )(src_r, dst_r, feat)


_BLK = _NPAD


def _t0_body(degp_ref, x_ref, d_ref, ci_ref, g0_ref):
    deg = jnp.sum(degp_ref[...], axis=0)
    degc = jnp.maximum(deg, 1.0)
    d = jnp.where(deg > 0, lax.rsqrt(degc), 0.0)
    d_ref[...] = d[:, None]
    ci_ref[...] = (1.0 / degc)[:, None]
    g0_ref[...] = x_ref[...] * d[:, None]


def _t0(degp, xp):
    return pl.pallas_call(
        _t0_body,
        grid=(_NPAD // _BLK,),
        in_specs=[
            pl.BlockSpec((_NW, _BLK), lambda i: (0, i)),
            pl.BlockSpec((_BLK, _DIN), lambda i: (i, 0)),
        ],
        out_specs=[
            pl.BlockSpec((_BLK, 1), lambda i: (i, 0)),
            pl.BlockSpec((_BLK, 1), lambda i: (i, 0)),
            pl.BlockSpec((_BLK, _DIN), lambda i: (i, 0)),
        ],
        out_shape=[
            jax.ShapeDtypeStruct((_NPAD, 1), jnp.float32),
            jax.ShapeDtypeStruct((_NPAD, 1), jnp.float32),
            jax.ShapeDtypeStruct((_NPAD, _DIN), jnp.float32),
        ],
    )(degp, xp)


def _prop_body(scale_g, a0_ref, a1_ref, d_ref, prev_ref, h_ref, g_ref):
    d = d_ref[...]
    h = d * (a0_ref[...] + a1_ref[...]) + prev_ref[...]
    h_ref[...] = h
    g_ref[...] = (d * h) if scale_g else h


def _prop(a, d, prev, scale_g):
    return pl.pallas_call(
        functools.partial(_prop_body, scale_g),
        grid=(_NPAD // _BLK,),
        in_specs=[pl.BlockSpec((_BLK, _DH), lambda i: (i, 0)),
                  pl.BlockSpec((_BLK, _DH), lambda i: (i, 0)),
                  pl.BlockSpec((_BLK, 1), lambda i: (i, 0)),
                  pl.BlockSpec((_BLK, _DH), lambda i: (i, 0))],
        out_specs=[pl.BlockSpec((_BLK, _DH), lambda i: (i, 0))] * 2,
        out_shape=[jax.ShapeDtypeStruct((_NPAD, _DH), jnp.float32)] * 2,
    )(a[0], a[1], d, prev)


def _selu(z):
    return _SELU_SCALE * jnp.where(z > 0, z, _SELU_ALPHA * (jnp.exp(z) - 1.0))


def _t3_body(s0_ref, s1_ref, ci_ref, h2_ref, w1lt_ref, b1_ref, w1rt_ref,
             b2_ref, w2rt_ref, y_ref, r_ref):
    m1 = (s0_ref[...] + s1_ref[...]) * ci_ref[...]
    h2 = h2_ref[...]
    z = (jnp.dot(m1, w1lt_ref[...], preferred_element_type=jnp.float32)
         + jnp.dot(h2, w1rt_ref[...], preferred_element_type=jnp.float32)
         + b1_ref[...])
    y = _selu(z)
    y_ref[...] = y
    r_ref[...] = (jnp.dot(y, w2rt_ref[...], preferred_element_type=jnp.float32)
                  + b2_ref[...])


def _t3(s0, s1, ci, h2, w1lt, b1, w1rt, b2, w2rt):
    full = lambda shape: pl.BlockSpec(shape, lambda i: tuple(0 for _ in shape))
    return pl.pallas_call(
        _t3_body,
        grid=(_NPAD // _BLK,),
        in_specs=[
            pl.BlockSpec((_BLK, _DH), lambda i: (i, 0)),
            pl.BlockSpec((_BLK, _DH), lambda i: (i, 0)),
            pl.BlockSpec((_BLK, 1), lambda i: (i, 0)),
            pl.BlockSpec((_BLK, _DH), lambda i: (i, 0)),
            full((_DIN, _DH)),
            full((1, _DH)),
            full((_DIN, _DH)),
            full((1, _DC)),
            full((_DH, _DC)),
        ],
        out_specs=[pl.BlockSpec((_BLK, _DH), lambda i: (i, 0)),
                   pl.BlockSpec((_BLK, _DC), lambda i: (i, 0))],
        out_shape=[jax.ShapeDtypeStruct((_NPAD, _DH), jnp.float32),
                   jax.ShapeDtypeStruct((_NPAD, _DC), jnp.float32)],
    )(s0, s1, ci, h2, w1lt, b1, w1rt, b2, w2rt)


def _t4_body(s0_ref, s1_ref, ci_ref, r_ref, w2lt_ref, o_ref):
    m2 = (s0_ref[...] + s1_ref[...]) * ci_ref[...]
    t = (jnp.dot(m2, w2lt_ref[...], preferred_element_type=jnp.float32)
         + r_ref[...])
    m = jnp.max(t, axis=1, keepdims=True)
    e = jnp.exp(t - m)
    o_ref[...] = e / jnp.sum(e, axis=1, keepdims=True)


def _t4(s0, s1, ci, r, w2lt):
    return pl.pallas_call(
        _t4_body,
        grid=(_NPAD // _BLK,),
        in_specs=[
            pl.BlockSpec((_BLK, _DH), lambda i: (i, 0)),
            pl.BlockSpec((_BLK, _DH), lambda i: (i, 0)),
            pl.BlockSpec((_BLK, 1), lambda i: (i, 0)),
            pl.BlockSpec((_BLK, _DC), lambda i: (i, 0)),
            pl.BlockSpec((_DH, _DC), lambda i: (0, 0)),
        ],
        out_specs=pl.BlockSpec((_BLK, _DC), lambda i: (i, 0)),
        out_shape=jax.ShapeDtypeStruct((_NPAD, _DC), jnp.float32),
    )(s0, s1, ci, r, w2lt)


def kernel(x, edge_index, W1l, b1, W1r, W2l, b2, W2r):
    src = edge_index[0]
    dst = edge_index[1]
    padv = jnp.full((_EPAD - _E,), _N, dtype=jnp.int32)
    src_r = jnp.concatenate([src, padv]).reshape(_NCHUNK, _B)
    dst_r = jnp.concatenate([dst, padv]).reshape(_NCHUNK, _B)
    xp = jnp.pad(x, ((0, _NPAD - _N), (0, 0)))

    zeros2d = jnp.zeros((80, 128), jnp.float32)

    degp = _deg_kernel(dst_r, zeros2d).reshape(_NW, _NPAD)
    d, ci, g0 = _t0(degp, xp)

    a1 = _seg_call(src_r, dst_r, g0, _DH)
    h1, g1 = _prop(a1, d, xp, True)
    a2 = _seg_call(src_r, dst_r, g1, _DH)
    h2, h2b = _prop(a2, d, h1, False)

    s1 = _seg_call(src_r, dst_r, h2b, _DH)
    y, r = _t3(s1[0], s1[1], ci, h2b,
               W1l.T, b1.reshape(1, _DH), W1r.T,
               b2.reshape(1, _DC), W2r.T)
    s2 = _seg_call(src_r, dst_r, y, _DH)
    out = _t4(s2[0], s2[1], ci, r, W2l.T)
    return out[:_N]
